# baseline (speedup 1.0000x reference)
import functools

import jax
import jax.numpy as jnp
from jax.experimental import pallas as pl
from jax.experimental.pallas import tpu as pltpu

_LANE = 128
_SUB = 8


def _ceil_to(n, m):
    return ((n + m - 1) // m) * m


def _fused_mlp_body(x_ref, w1_hbm, b1_ref, w2_hbm, b2_ref, w3_hbm, b3_ref,
                    o_ref, w1f, w2f, w3f, w1b, w2b, w3b, sems):
    j = pl.program_id(0)
    In_p = w1_hbm.shape[0]
    half = (In_p // 2) if In_p >= 256 else In_p
    lo = pl.ds(0, half)
    hi = pl.ds(half, In_p - half)

    @pl.when(j == 0)
    def _ramp_step():
        pltpu.make_async_copy(w1_hbm.at[lo], w1f.at[lo], sems.at[0]).start()
        if In_p > half:
            pltpu.make_async_copy(w1_hbm.at[hi], w1f.at[hi], sems.at[3]).start()
        pltpu.make_async_copy(w2_hbm, w2f, sems.at[1]).start()
        pltpu.make_async_copy(w3_hbm, w3f, sems.at[2]).start()
        xb = x_ref[...].astype(jnp.bfloat16)
        pltpu.make_async_copy(w1_hbm.at[lo], w1f.at[lo], sems.at[0]).wait()
        w1b[lo] = w1f[lo].astype(jnp.bfloat16)
        h = jnp.dot(xb[:, :half], w1b[lo],
                    preferred_element_type=jnp.float32)
        if In_p > half:
            pltpu.make_async_copy(w1_hbm.at[hi], w1f.at[hi], sems.at[3]).wait()
            w1b[hi] = w1f[hi].astype(jnp.bfloat16)
            h = h + jnp.dot(xb[:, half:], w1b[hi],
                            preferred_element_type=jnp.float32)
        h = jnp.maximum(h + b1_ref[...], 0.0).astype(jnp.bfloat16)
        pltpu.make_async_copy(w2_hbm, w2f, sems.at[1]).wait()
        w2b[...] = w2f[...].astype(jnp.bfloat16)
        g = jnp.dot(h, w2b[...], preferred_element_type=jnp.float32)
        g = jnp.maximum(g + b2_ref[...], 0.0).astype(jnp.bfloat16)
        pltpu.make_async_copy(w3_hbm, w3f, sems.at[2]).wait()
        w3b[...] = w3f[...].astype(jnp.bfloat16)
        y = jnp.dot(g, w3b[...], preferred_element_type=jnp.float32)
        o_ref[...] = (y + b3_ref[...]).astype(o_ref.dtype)

    @pl.when(j != 0)
    def _steady_step():
        xb = x_ref[...].astype(jnp.bfloat16)
        h = jnp.dot(xb, w1b[...], preferred_element_type=jnp.float32)
        h = jnp.maximum(h + b1_ref[...], 0.0).astype(jnp.bfloat16)
        g = jnp.dot(h, w2b[...], preferred_element_type=jnp.float32)
        g = jnp.maximum(g + b2_ref[...], 0.0).astype(jnp.bfloat16)
        y = jnp.dot(g, w3b[...], preferred_element_type=jnp.float32)
        o_ref[...] = (y + b3_ref[...]).astype(o_ref.dtype)


@jax.jit
def _fused_mlp(x, w1, b1, w2, b2, w3, b3):
    B, In = x.shape
    H = w1.shape[1]
    C = w3.shape[1]
    In_p = _ceil_to(In, _LANE)
    H_p = _ceil_to(H, _LANE)
    C_p = _ceil_to(C, _LANE)

    TB = min(1024, _ceil_to(B, _SUB))
    B_p = _ceil_to(B, TB)
    nb = B_p // TB

    def pad_to(a, r, c):
        if a.shape == (r, c):
            return a
        return jnp.pad(a, ((0, r - a.shape[0]), (0, c - a.shape[1])))

    x_p = pad_to(x, B_p, In_p)
    w1_p = pad_to(w1, In_p, H_p)
    w2_p = pad_to(w2, H_p, H_p)
    w3_p = pad_to(w3, H_p, C_p)
    b1_p = pad_to(b1.reshape(1, H), 1, H_p)
    b2_p = pad_to(b2.reshape(1, H), 1, H_p)
    b3_p = pad_to(b3.reshape(1, C), 1, C_p)

    out_p = pl.pallas_call(
        _fused_mlp_body,
        out_shape=jax.ShapeDtypeStruct((B_p, C_p), x.dtype),
        grid=(nb,),
        in_specs=[
            pl.BlockSpec((TB, In_p), lambda j: (j, 0)),
            pl.BlockSpec(memory_space=pl.ANY),
            pl.BlockSpec((1, H_p), lambda j: (0, 0)),
            pl.BlockSpec(memory_space=pl.ANY),
            pl.BlockSpec((1, H_p), lambda j: (0, 0)),
            pl.BlockSpec(memory_space=pl.ANY),
            pl.BlockSpec((1, C_p), lambda j: (0, 0)),
        ],
        out_specs=pl.BlockSpec((TB, C_p), lambda j: (j, 0)),
        scratch_shapes=[
            pltpu.VMEM((In_p, H_p), jnp.float32),
            pltpu.VMEM((H_p, H_p), jnp.float32),
            pltpu.VMEM((H_p, C_p), jnp.float32),
            pltpu.VMEM((In_p, H_p), jnp.bfloat16),
            pltpu.VMEM((H_p, H_p), jnp.bfloat16),
            pltpu.VMEM((H_p, C_p), jnp.bfloat16),
            pltpu.SemaphoreType.DMA((4,)),
        ],
        compiler_params=pltpu.CompilerParams(
            dimension_semantics=("arbitrary",),
            vmem_limit_bytes=64 << 20,
        ),
    )(x_p, w1_p, b1_p, w2_p, b2_p, w3_p, b3_p)

    if (B_p, C_p) == (B, C):
        return out_p
    return out_p[:B, :C]


def kernel(x, w1, b1, w2, b2, w3, b3):
    return _fused_mlp(x, w1, b1, w2, b2, w3, b3)


# speedup vs baseline: 1.0025x; 1.0025x over previous
import jax
import jax.numpy as jnp
from jax.experimental import pallas as pl
from jax.experimental.pallas import tpu as pltpu

_LANE = 128
_SUB = 8


def _ceil_to(n, m):
    return ((n + m - 1) // m) * m


def _fused_mlp_body(x_ref, w1_hbm, b1_ref, w2_hbm, b2_ref, w3_hbm, b3_ref,
                    o_ref, w1f, w2f, w3f, w1b, w2b, w3b, sems):
    j = pl.program_id(0)

    @pl.when(j == 0)
    def _ramp_step():
        pltpu.make_async_copy(w1_hbm, w1f, sems.at[0]).start()
        pltpu.make_async_copy(w2_hbm, w2f, sems.at[1]).start()
        pltpu.make_async_copy(w3_hbm, w3f, sems.at[2]).start()
        pltpu.make_async_copy(w1_hbm, w1f, sems.at[0]).wait()
        w1b[...] = w1f[...].astype(jnp.bfloat16)
        xb = x_ref[...].astype(jnp.bfloat16)
        h = jnp.dot(xb, w1b[...], preferred_element_type=jnp.float32)
        h = jnp.maximum(h + b1_ref[...], 0.0).astype(jnp.bfloat16)
        pltpu.make_async_copy(w2_hbm, w2f, sems.at[1]).wait()
        w2b[...] = w2f[...].astype(jnp.bfloat16)
        g = jnp.dot(h, w2b[...], preferred_element_type=jnp.float32)
        g = jnp.maximum(g + b2_ref[...], 0.0).astype(jnp.bfloat16)
        pltpu.make_async_copy(w3_hbm, w3f, sems.at[2]).wait()
        w3b[...] = w3f[...].astype(jnp.bfloat16)
        y = jnp.dot(g, w3b[...], preferred_element_type=jnp.float32)
        o_ref[...] = (y + b3_ref[...]).astype(o_ref.dtype)

    @pl.when(j != 0)
    def _steady_step():
        xb = x_ref[...].astype(jnp.bfloat16)
        h = jnp.dot(xb, w1b[...], preferred_element_type=jnp.float32)
        h = jnp.maximum(h + b1_ref[...], 0.0).astype(jnp.bfloat16)
        g = jnp.dot(h, w2b[...], preferred_element_type=jnp.float32)
        g = jnp.maximum(g + b2_ref[...], 0.0).astype(jnp.bfloat16)
        y = jnp.dot(g, w3b[...], preferred_element_type=jnp.float32)
        o_ref[...] = (y + b3_ref[...]).astype(o_ref.dtype)


@jax.jit
def _fused_mlp(x, w1, b1, w2, b2, w3, b3):
    B, In = x.shape
    H = w1.shape[1]
    C = w3.shape[1]
    In_p = _ceil_to(In, _LANE)
    H_p = _ceil_to(H, _LANE)
    C_p = _ceil_to(C, _LANE)

    TB = min(1024, _ceil_to(B, _SUB))
    B_p = _ceil_to(B, TB)
    nb = B_p // TB

    def pad_to(a, r, c):
        if a.shape == (r, c):
            return a
        return jnp.pad(a, ((0, r - a.shape[0]), (0, c - a.shape[1])))

    x_p = pad_to(x, B_p, In_p)
    w1_p = pad_to(w1, In_p, H_p)
    w2_p = pad_to(w2, H_p, H_p)
    w3_p = pad_to(w3, H_p, C_p)
    b1_p = pad_to(b1.reshape(1, H), 1, H_p)
    b2_p = pad_to(b2.reshape(1, H), 1, H_p)
    b3_p = pad_to(b3.reshape(1, C), 1, C_p)

    out_p = pl.pallas_call(
        _fused_mlp_body,
        out_shape=jax.ShapeDtypeStruct((B_p, C_p), x.dtype),
        grid=(nb,),
        in_specs=[
            pl.BlockSpec((TB, In_p), lambda j: (j, 0)),
            pl.BlockSpec(memory_space=pl.ANY),
            pl.BlockSpec((1, H_p), lambda j: (0, 0)),
            pl.BlockSpec(memory_space=pl.ANY),
            pl.BlockSpec((1, H_p), lambda j: (0, 0)),
            pl.BlockSpec(memory_space=pl.ANY),
            pl.BlockSpec((1, C_p), lambda j: (0, 0)),
        ],
        out_specs=pl.BlockSpec((TB, C_p), lambda j: (j, 0)),
        scratch_shapes=[
            pltpu.VMEM((In_p, H_p), jnp.float32),
            pltpu.VMEM((H_p, H_p), jnp.float32),
            pltpu.VMEM((H_p, C_p), jnp.float32),
            pltpu.VMEM((In_p, H_p), jnp.bfloat16),
            pltpu.VMEM((H_p, H_p), jnp.bfloat16),
            pltpu.VMEM((H_p, C_p), jnp.bfloat16),
            pltpu.SemaphoreType.DMA((3,)),
        ],
        compiler_params=pltpu.CompilerParams(
            dimension_semantics=("arbitrary",),
            vmem_limit_bytes=64 << 20,
        ),
    )(x_p, w1_p, b1_p, w2_p, b2_p, w3_p, b3_p)

    if (B_p, C_p) == (B, C):
        return out_p
    return out_p[:B, :C]


def kernel(x, w1, b1, w2, b2, w3, b3):
    return _fused_mlp(x, w1, b1, w2, b2, w3, b3)


# speedup vs baseline: 1.0061x; 1.0036x over previous
import jax
import jax.numpy as jnp
from jax.experimental import pallas as pl
from jax.experimental.pallas import tpu as pltpu

_LANE = 128
_SUB = 8


def _ceil_to(n, m):
    return ((n + m - 1) // m) * m


def _fused_mlp_body(x_ref, w1_hbm, b1_ref, w2_hbm, b2_ref, w3_hbm, b3_ref,
                    o_ref, w1f, w2f, w3f, w1b, w2b, w3b, sems):
    j = pl.program_id(0)

    @pl.when(j == 0)
    def _ramp_step():
        pltpu.make_async_copy(w1_hbm, w1f, sems.at[0]).start()
        pltpu.make_async_copy(w2_hbm, w2f, sems.at[1]).start()
        pltpu.make_async_copy(w3_hbm, w3f, sems.at[2]).start()
        xb = x_ref[...].astype(jnp.bfloat16)
        pltpu.make_async_copy(w1_hbm, w1f, sems.at[0]).wait()
        w1b[...] = w1f[...].astype(jnp.bfloat16)
        h = jnp.dot(xb, w1b[...], preferred_element_type=jnp.float32)
        h = jnp.maximum(h + b1_ref[...], 0.0).astype(jnp.bfloat16)
        pltpu.make_async_copy(w2_hbm, w2f, sems.at[1]).wait()
        w2b[...] = w2f[...].astype(jnp.bfloat16)
        g = jnp.dot(h, w2b[...], preferred_element_type=jnp.float32)
        g = jnp.maximum(g + b2_ref[...], 0.0).astype(jnp.bfloat16)
        pltpu.make_async_copy(w3_hbm, w3f, sems.at[2]).wait()
        w3b[...] = w3f[...].astype(jnp.bfloat16)
        y = jnp.dot(g, w3b[...], preferred_element_type=jnp.float32)
        o_ref[...] = (y + b3_ref[...]).astype(o_ref.dtype)

    @pl.when(j != 0)
    def _steady_step():
        xb = x_ref[...].astype(jnp.bfloat16)
        h = jnp.dot(xb, w1b[...], preferred_element_type=jnp.float32)
        h = jnp.maximum(h + b1_ref[...], 0.0).astype(jnp.bfloat16)
        g = jnp.dot(h, w2b[...], preferred_element_type=jnp.float32)
        g = jnp.maximum(g + b2_ref[...], 0.0).astype(jnp.bfloat16)
        y = jnp.dot(g, w3b[...], preferred_element_type=jnp.float32)
        o_ref[...] = (y + b3_ref[...]).astype(o_ref.dtype)


@jax.jit
def _fused_mlp(x, w1, b1, w2, b2, w3, b3):
    B, In = x.shape
    H = w1.shape[1]
    C = w3.shape[1]
    In_p = _ceil_to(In, _LANE)
    H_p = _ceil_to(H, _LANE)
    C_p = _ceil_to(C, _LANE)

    TB = min(1024, _ceil_to(B, _SUB))
    B_p = _ceil_to(B, TB)
    nb = B_p // TB

    def pad_to(a, r, c):
        if a.shape == (r, c):
            return a
        return jnp.pad(a, ((0, r - a.shape[0]), (0, c - a.shape[1])))

    x_p = pad_to(x, B_p, In_p)
    w1_p = pad_to(w1, In_p, H_p)
    w2_p = pad_to(w2, H_p, H_p)
    w3_p = pad_to(w3, H_p, C_p)
    b1_p = pad_to(b1.reshape(1, H), 1, H_p)
    b2_p = pad_to(b2.reshape(1, H), 1, H_p)
    b3_p = pad_to(b3.reshape(1, C), 1, C_p)

    out_p = pl.pallas_call(
        _fused_mlp_body,
        out_shape=jax.ShapeDtypeStruct((B_p, C_p), x.dtype),
        grid=(nb,),
        in_specs=[
            pl.BlockSpec((TB, In_p), lambda j: (j, 0)),
            pl.BlockSpec(memory_space=pl.ANY),
            pl.BlockSpec((1, H_p), lambda j: (0, 0)),
            pl.BlockSpec(memory_space=pl.ANY),
            pl.BlockSpec((1, H_p), lambda j: (0, 0)),
            pl.BlockSpec(memory_space=pl.ANY),
            pl.BlockSpec((1, C_p), lambda j: (0, 0)),
        ],
        out_specs=pl.BlockSpec((TB, C_p), lambda j: (j, 0)),
        scratch_shapes=[
            pltpu.VMEM((In_p, H_p), jnp.float32),
            pltpu.VMEM((H_p, H_p), jnp.float32),
            pltpu.VMEM((H_p, C_p), jnp.float32),
            pltpu.VMEM((In_p, H_p), jnp.bfloat16),
            pltpu.VMEM((H_p, H_p), jnp.bfloat16),
            pltpu.VMEM((H_p, C_p), jnp.bfloat16),
            pltpu.SemaphoreType.DMA((3,)),
        ],
        compiler_params=pltpu.CompilerParams(
            dimension_semantics=("arbitrary",),
            vmem_limit_bytes=64 << 20,
        ),
    )(x_p, w1_p, b1_p, w2_p, b2_p, w3_p, b3_p)

    if (B_p, C_p) == (B, C):
        return out_p
    return out_p[:B, :C]


def kernel(x, w1, b1, w2, b2, w3, b3):
    return _fused_mlp(x, w1, b1, w2, b2, w3, b3)
